# baseline (speedup 1.0000x reference)
import functools

import jax
import jax.numpy as jnp
from jax import lax
from jax.experimental import pallas as pl
from jax.experimental.pallas import tpu as pltpu
from jax.experimental.pallas import tpu_sc as plsc

D = 64
SCALE = 8.0

B_TOTAL = 4096 * 200
NW = 32
B_PER_W = B_TOTAL // NW
CHUNK = 512
GSZ = 128
N_GATHER = CHUNK // GSZ
N_CHUNKS = B_PER_W // CHUNK


def _make_kernel():
  mesh = plsc.VectorSubcoreMesh(core_axis_name="c", subcore_axis_name="s")

  @functools.partial(
      pl.kernel,
      mesh=mesh,
      compiler_params=pltpu.CompilerParams(use_tc_tiling_on_sc=False),
      out_type=jax.ShapeDtypeStruct((B_TOTAL, D), jnp.float32),
      scratch_types=[
          pltpu.VMEM((CHUNK,), jnp.int32),
          pltpu.VMEM((CHUNK, D), jnp.float32),
          pltpu.SemaphoreType.DMA,
      ],
  )
  def emb_kernel(x_hbm, table_hbm, out_hbm, idx_v, rows_v, sem):
    wid = lax.axis_index("s") * 2 + lax.axis_index("c")
    wbase = wid * B_PER_W

    def chunk_body(g, carry):
      base = wbase + g * CHUNK
      pltpu.sync_copy(x_hbm.at[pl.ds(base, CHUNK)], idx_v)
      copies = [
          pltpu.async_copy(
              table_hbm.at[idx_v.at[pl.ds(j * GSZ, GSZ)]],
              rows_v.at[pl.ds(j * GSZ, GSZ)],
              sem,
          )
          for j in range(N_GATHER)
      ]
      for c in copies:
        c.wait()

      def scale_row(r, c2):
        for j in range(D // 16):
          rows_v[r, pl.ds(j * 16, 16)] = rows_v[r, pl.ds(j * 16, 16)] * SCALE
        return c2

      lax.fori_loop(0, CHUNK, scale_row, 0, unroll=2)

      pltpu.sync_copy(rows_v, out_hbm.at[pl.ds(base, CHUNK)])
      return carry

    lax.fori_loop(0, N_CHUNKS, chunk_body, 0)

  return emb_kernel


_emb = _make_kernel()


@jax.jit
def kernel(x, table):
  xf = x.reshape(-1).astype(jnp.int32)
  out = _emb(xf, table)
  return out.reshape(x.shape[0], x.shape[1], D)


# speedup vs baseline: 1.0889x; 1.0889x over previous
import functools

import jax
import jax.numpy as jnp
from jax import lax
from jax.experimental import pallas as pl
from jax.experimental.pallas import tpu as pltpu
from jax.experimental.pallas import tpu_sc as plsc

D = 64
SCALE = 8.0

B_TOTAL = 4096 * 200
NW = 32
B_PER_W = B_TOTAL // NW
CHUNK = 512
GSZ = 128
N_GATHER = CHUNK // GSZ
N_CHUNKS = B_PER_W // CHUNK
NBUF = 2


def _make_kernel():
  mesh = plsc.VectorSubcoreMesh(core_axis_name="c", subcore_axis_name="s")

  @functools.partial(
      pl.kernel,
      mesh=mesh,
      compiler_params=pltpu.CompilerParams(use_tc_tiling_on_sc=False),
      out_type=jax.ShapeDtypeStruct((B_TOTAL, D), jnp.float32),
      scratch_types=[
          pltpu.VMEM((NBUF, CHUNK), jnp.int32),
          pltpu.VMEM((NBUF, CHUNK, D), jnp.float32),
          pltpu.SemaphoreType.DMA,
          pltpu.SemaphoreType.DMA,
          pltpu.SemaphoreType.DMA,
          pltpu.SemaphoreType.DMA,
          pltpu.SemaphoreType.DMA,
          pltpu.SemaphoreType.DMA,
      ],
  )
  def emb_kernel(x_hbm, table_hbm, out_hbm, idx_v, rows_v,
                 si0, si1, sg0, sg1, so0, so1):
    sem_i = (si0, si1)
    sem_g = (sg0, sg1)
    sem_o = (so0, so1)
    wid = lax.axis_index("s") * 2 + lax.axis_index("c")
    wbase = wid * B_PER_W

    def idx_copy(g, b):
      return pltpu.make_async_copy(
          x_hbm.at[pl.ds(wbase + g * CHUNK, CHUNK)], idx_v.at[b], sem_i[b])

    def fire_gather(b):
      for j in range(N_GATHER):
        pltpu.async_copy(
            table_hbm.at[idx_v.at[b, pl.ds(j * GSZ, GSZ)]],
            rows_v.at[b, pl.ds(j * GSZ, GSZ)],
            sem_g[b])

    def drain_gather(b):
      pltpu.make_async_copy(
          out_hbm.at[pl.ds(0, CHUNK)], rows_v.at[b], sem_g[b]).wait()

    def out_copy(g, b):
      return pltpu.make_async_copy(
          rows_v.at[b], out_hbm.at[pl.ds(wbase + g * CHUNK, CHUNK)], sem_o[b])

    def scale(b):
      def scale_row(r, c):
        for j in range(D // 16):
          rows_v[b, r, pl.ds(j * 16, 16)] = (
              rows_v[b, r, pl.ds(j * 16, 16)] * SCALE)
        return c

      lax.fori_loop(0, CHUNK, scale_row, 0, unroll=4)

    idx_copy(0, 0).start()
    idx_copy(1, 1).start()
    idx_copy(0, 0).wait()
    fire_gather(0)

    @pl.loop(0, N_CHUNKS, step=NBUF)
    def pipeline(g0):
      for b in range(NBUF):
        g = g0 + b
        nb = (b + 1) % NBUF

        @pl.when(g + 1 < N_CHUNKS)
        def _():
          idx_copy(g + 1, nb).wait()

          @pl.when(g >= 1)
          def _():
            out_copy(g - 1, nb).wait()

          fire_gather(nb)

        drain_gather(b)

        @pl.when(g + 2 < N_CHUNKS)
        def _():
          idx_copy(g + 2, b).start()

        scale(b)
        out_copy(g, b).start()

    out_copy(N_CHUNKS - 1, (N_CHUNKS - 1) % NBUF).wait()

  return emb_kernel


_emb = _make_kernel()


@jax.jit
def kernel(x, table):
  xf = x.reshape(-1).astype(jnp.int32)
  out = _emb(xf, table)
  return out.reshape(x.shape[0], x.shape[1], D)


# speedup vs baseline: 1.5509x; 1.4243x over previous
import functools

import jax
import jax.numpy as jnp
from jax import lax
from jax.experimental import pallas as pl
from jax.experimental.pallas import tpu as pltpu
from jax.experimental.pallas import tpu_sc as plsc

D = 64
SCALE = 8.0

B_TOTAL = 4096 * 200
NW = 32
B_PER_W = B_TOTAL // NW
CHUNK = 512
GSZ = 128
N_GATHER = CHUNK // GSZ
N_CHUNKS = B_PER_W // CHUNK
NBUF = 2


def _make_kernel():
  mesh = plsc.VectorSubcoreMesh(core_axis_name="c", subcore_axis_name="s")

  @functools.partial(
      pl.kernel,
      mesh=mesh,
      compiler_params=pltpu.CompilerParams(use_tc_tiling_on_sc=False),
      out_type=jax.ShapeDtypeStruct((B_TOTAL, 2 * D), jnp.float32),
      scratch_types=[
          pltpu.VMEM((NBUF, CHUNK), jnp.int32),
          pltpu.VMEM((NBUF, CHUNK, D), jnp.float32),
          pltpu.SemaphoreType.DMA,
          pltpu.SemaphoreType.DMA,
          pltpu.SemaphoreType.DMA,
          pltpu.SemaphoreType.DMA,
          pltpu.SemaphoreType.DMA,
          pltpu.SemaphoreType.DMA,
      ],
  )
  def emb_kernel(x_hbm, table_hbm, out_hbm, idx_v, rows_v,
                 si0, si1, sg0, sg1, so0, so1):
    sem_i = (si0, si1)
    sem_g = (sg0, sg1)
    sem_o = (so0, so1)
    wid = lax.axis_index("s") * 2 + lax.axis_index("c")
    wbase = wid * B_PER_W

    def idx_copy(g, b):
      return pltpu.make_async_copy(
          x_hbm.at[pl.ds(wbase + g * CHUNK, CHUNK)], idx_v.at[b], sem_i[b])

    def fire_gather(b):
      for j in range(N_GATHER):
        pltpu.async_copy(
            table_hbm.at[idx_v.at[b, pl.ds(j * GSZ, GSZ)]],
            rows_v.at[b, pl.ds(j * GSZ, GSZ)],
            sem_g[b])

    def drain_gather(b):
      pltpu.make_async_copy(
          out_hbm.at[pl.ds(0, CHUNK), pl.ds(0, D)], rows_v.at[b],
          sem_g[b]).wait()

    def out_copy(g, b):
      return pltpu.make_async_copy(
          rows_v.at[b],
          out_hbm.at[pl.ds(wbase + g * CHUNK, CHUNK), pl.ds(0, D)],
          sem_o[b])

    def scale(b):
      def scale_row(r, c):
        for j in range(D // 16):
          rows_v[b, r, pl.ds(j * 16, 16)] = (
              rows_v[b, r, pl.ds(j * 16, 16)] * SCALE)
        return c

      lax.fori_loop(0, CHUNK, scale_row, 0, unroll=4)

    idx_copy(0, 0).start()
    idx_copy(1, 1).start()
    idx_copy(0, 0).wait()
    fire_gather(0)

    @pl.loop(0, N_CHUNKS, step=NBUF)
    def pipeline(g0):
      for b in range(NBUF):
        g = g0 + b
        nb = (b + 1) % NBUF

        @pl.when(g + 1 < N_CHUNKS)
        def _():
          idx_copy(g + 1, nb).wait()

          @pl.when(g >= 1)
          def _():
            out_copy(g - 1, nb).wait()

          fire_gather(nb)

        drain_gather(b)

        @pl.when(g + 2 < N_CHUNKS)
        def _():
          idx_copy(g + 2, b).start()

        scale(b)
        out_copy(g, b).start()

    out_copy(N_CHUNKS - 1, (N_CHUNKS - 1) % NBUF).wait()

  return emb_kernel


_emb = _make_kernel()


@jax.jit
def kernel(x, table):
  xf = x.reshape(-1) * 2
  t2 = jnp.pad(table, ((0, 0), (0, D))).reshape(2 * 1_000_000, D)
  out = _emb(xf, t2)
  return out[:, :D].reshape(x.shape[0], x.shape[1], D)


# speedup vs baseline: 1.9681x; 1.2690x over previous
import functools

import jax
import jax.numpy as jnp
from jax import lax
from jax.experimental import pallas as pl
from jax.experimental.pallas import tpu as pltpu
from jax.experimental.pallas import tpu_sc as plsc

D = 64
SCALE = 8.0

B_TOTAL = 4096 * 200
NW = 32
B_PER_W = B_TOTAL // NW
CHUNK = 512
GSZ = 128
N_GATHER = CHUNK // GSZ
N_CHUNKS = B_PER_W // CHUNK
NBUF = 2


def _make_kernel():
  mesh = plsc.VectorSubcoreMesh(core_axis_name="c", subcore_axis_name="s")

  @functools.partial(
      pl.kernel,
      mesh=mesh,
      compiler_params=pltpu.CompilerParams(use_tc_tiling_on_sc=False),
      out_type=jax.ShapeDtypeStruct((B_TOTAL, 2 * D), jnp.float32),
      scratch_types=[
          pltpu.VMEM((NBUF, CHUNK), jnp.int32),
          pltpu.VMEM((NBUF, CHUNK, D), jnp.float32),
          pltpu.SemaphoreType.DMA,
          pltpu.SemaphoreType.DMA,
          pltpu.SemaphoreType.DMA,
          pltpu.SemaphoreType.DMA,
          pltpu.SemaphoreType.DMA,
          pltpu.SemaphoreType.DMA,
      ],
  )
  def emb_kernel(x_hbm, table_hbm, out_hbm, idx_v, rows_v,
                 si0, si1, sg0, sg1, so0, so1):
    sem_i = (si0, si1)
    sem_g = (sg0, sg1)
    sem_o = (so0, so1)
    wid = lax.axis_index("s") * 2 + lax.axis_index("c")
    wbase = wid * B_PER_W

    def idx_copy(g, b):
      return pltpu.make_async_copy(
          x_hbm.at[pl.ds(wbase + g * CHUNK, CHUNK)], idx_v.at[b], sem_i[b])

    def fire_gather(b):
      for j in range(N_GATHER):
        pltpu.async_copy(
            table_hbm.at[idx_v.at[b, pl.ds(j * GSZ, GSZ)]],
            rows_v.at[b, pl.ds(j * GSZ, GSZ)],
            sem_g[b])

    def drain_gather(b):
      pltpu.make_async_copy(
          out_hbm.at[pl.ds(0, CHUNK), pl.ds(0, D)], rows_v.at[b],
          sem_g[b]).wait()

    def out_copy(g, b):
      return pltpu.make_async_copy(
          rows_v.at[b],
          out_hbm.at[pl.ds(wbase + g * CHUNK, CHUNK), pl.ds(0, D)],
          sem_o[b])

    def scale(b):
      def scale_row(r, c):
        for j in range(D // 16):
          rows_v[b, r, pl.ds(j * 16, 16)] = (
              rows_v[b, r, pl.ds(j * 16, 16)] * SCALE)
        return c

      lax.fori_loop(0, CHUNK, scale_row, 0, unroll=4)

    idx_copy(0, 0).start()
    idx_copy(1, 1).start()
    idx_copy(0, 0).wait()
    fire_gather(0)

    @pl.loop(0, N_CHUNKS, step=NBUF)
    def pipeline(g0):
      for b in range(NBUF):
        g = g0 + b
        nb = (b + 1) % NBUF

        @pl.when(g + 1 < N_CHUNKS)
        def _():
          idx_copy(g + 1, nb).wait()

          @pl.when(g >= 1)
          def _():
            out_copy(g - 1, nb).wait()

          fire_gather(nb)

        drain_gather(b)

        @pl.when(g + 2 < N_CHUNKS)
        def _():
          idx_copy(g + 2, b).start()

        scale(b)
        out_copy(g, b).start()

    out_copy(N_CHUNKS - 1, (N_CHUNKS - 1) % NBUF).wait()

  return emb_kernel


_emb = _make_kernel()

V = 1_000_000
TBW = 4096
N_TBLK = (V + TBW - 1) // TBW


def _transpose_pad_kernel(tt_ref, out_ref):
  xt = tt_ref[...].T
  out_ref[...] = jnp.concatenate(
      [xt, jnp.zeros((TBW, D), jnp.float32)], axis=1)


_tpad = pl.pallas_call(
    _transpose_pad_kernel,
    grid=(N_TBLK,),
    in_specs=[pl.BlockSpec((D, TBW), lambda i: (0, i))],
    out_specs=pl.BlockSpec((TBW, 2 * D), lambda i: (i, 0)),
    out_shape=jax.ShapeDtypeStruct((V, 2 * D), jnp.float32),
)


@jax.jit
def kernel(x, table):
  t128 = _tpad(table.T)
  t2 = t128.reshape(2 * V, D)
  xf = x.reshape(-1) * 2
  out = _emb(xf, t2)
  return out[:, :D].reshape(x.shape[0], x.shape[1], D)
